# baseline (speedup 1.0000x reference)
import functools

import jax
import jax.numpy as jnp
from jax.experimental import pallas as pl

N = 10000
E1 = 320000
D1 = 100


def _m_body(f_ref, s_ref, o_ref):
    f = f_ref[...]
    s = s_ref[...]
    sig = 1.0 / (1.0 + jnp.exp(-f))
    sp = jnp.maximum(s, 0.0) + jnp.log1p(jnp.exp(-jnp.abs(s)))
    o_ref[...] = sig * sp


def _edge_m(f, s):
    e = f.shape[0]
    blk = 8000
    assert e % blk == 0
    grid = e // blk
    return pl.pallas_call(
        _m_body,
        grid=(grid,),
        in_specs=[
            pl.BlockSpec((blk, D1), lambda i: (i, 0)),
            pl.BlockSpec((blk, D1), lambda i: (i, 0)),
        ],
        out_specs=pl.BlockSpec((blk, D1), lambda i: (i, 0)),
        out_shape=jax.ShapeDtypeStruct((e, D1), jnp.float32),
    )(f, s)


def _conv(xc, src, dst, ea, Wf, bf, Ws, bs):
    A = xc @ Wf[:D1]
    B = xc @ Wf[D1:2 * D1]
    C = xc @ Ws[:D1]
    D = xc @ Ws[D1:2 * D1]
    f = A[dst] + B[src] + ea @ Wf[2 * D1:] + bf
    s = C[dst] + D[src] + ea @ Ws[2 * D1:] + bs
    m = _edge_m(f, s)
    agg = jax.ops.segment_sum(m, dst, num_segments=N)
    cnt = jax.ops.segment_sum(jnp.ones((dst.shape[0],), jnp.float32), dst,
                              num_segments=N)
    return xc + agg / jnp.maximum(cnt, 1.0)[:, None]


def _bn(h, g, b):
    mu = jnp.mean(h, axis=0)
    v = jnp.var(h, axis=0)
    return (h - mu) / jnp.sqrt(v + 1e-5) * g + b


def kernel(x, edge_index_1, edge_index_2, edge_index_3, edge_attr_1, edge_attr_2, edge_attr_3, batch, parameters, pre_W0, pre_b0, pre_W1, pre_b1, pre_W2, pre_b2, pre_W3, pre_b3, c1_Wf, c1_bf, c1_Ws, c1_bs, c12_Wf, c12_bf, c12_Ws, c12_bs, c13_Wf, c13_bf, c13_Ws, c13_bs, c14_Wf, c14_bf, c14_Ws, c14_bs, c24_Wf, c24_bf, c24_Ws, c24_bs, c34_Wf, c34_bf, c34_Ws, c34_bs, pool_W, pool_b, bn1_g, bn1_b, bn12_g, bn12_b, bn13_g, bn13_b, bn14_g, bn14_b, bn24_g, bn24_b, bn34_g, bn34_b, g1_W, g1_b, g2_W, g2_b):
    h = x
    for W, B in ((pre_W0, pre_b0), (pre_W1, pre_b1), (pre_W2, pre_b2),
                 (pre_W3, pre_b3)):
        h = jax.nn.relu(h @ W + B)

    G = parameters.shape[0]
    scores = []
    for ei, ea, Wf, bf, Ws, bs, g, b in (
        (edge_index_1, edge_attr_1, c1_Wf, c1_bf, c1_Ws, c1_bs, bn1_g, bn1_b),
        (edge_index_2, edge_attr_2, c12_Wf, c12_bf, c12_Ws, c12_bs, bn12_g, bn12_b),
        (edge_index_3, edge_attr_3, c13_Wf, c13_bf, c13_Ws, c13_bs, bn13_g, bn13_b),
    ):
        src, dst = ei[0], ei[1]
        xi = _bn(_conv(h, src, dst, ea, Wf, bf, Ws, bs), g, b)
        a = xi @ pool_W[:D1, 0]
        bb = xi @ pool_W[D1:2 * D1, 0]
        c = ea @ pool_W[2 * D1:, 0]
        score = jax.nn.sigmoid(a[src] + bb[dst] + c + pool_b[0])
        scores.append(score)

    ei = jnp.concatenate([edge_index_1, edge_index_2, edge_index_3], axis=1)
    ea4 = jnp.concatenate([
        jnp.concatenate([edge_attr_1, scores[0][:, None]], axis=1),
        jnp.concatenate([edge_attr_2, scores[1][:, None]], axis=1),
        jnp.concatenate([edge_attr_3, scores[2][:, None]], axis=1),
    ], axis=0)
    src, dst = ei[0], ei[1]

    out = h
    for Wf, bf, Ws, bs, g, b in (
        (c14_Wf, c14_bf, c14_Ws, c14_bs, bn14_g, bn14_b),
        (c24_Wf, c24_bf, c24_Ws, c24_bs, bn24_g, bn24_b),
        (c34_Wf, c34_bf, c34_Ws, c34_bs, bn34_g, bn34_b),
    ):
        out = _bn(_conv(out, src, dst, ea4, Wf, bf, Ws, bs), g, b)

    pars = jnp.reshape(parameters, (-1, 2))
    ps = jax.ops.segment_sum(out, batch, num_segments=G)
    cnt = jax.ops.segment_sum(jnp.ones((out.shape[0],), jnp.float32), batch,
                              num_segments=G)
    pooled = ps / jnp.maximum(cnt, 1.0)[:, None]
    o = jnp.concatenate([pooled, pars], axis=1)
    o = o @ g1_W + g1_b
    o = o @ g2_W + g2_b
    return o.reshape(-1)


# speedup vs baseline: 4.5219x; 4.5219x over previous
import functools

import jax
import jax.numpy as jnp
from jax import lax
from jax.experimental import pallas as pl
from jax.experimental.pallas import tpu as pltpu
from jax.experimental.pallas import tpu_sc as plsc

N = 10000
NPAD = 10240
D1 = 100
DT = 128
_NC, _NS = 2, 16
_NW = _NC * _NS


def _sc_gather(table, idx, d, tc_tiling=True):
    b = idx.shape[0]
    bpw = b // _NW
    ch = 400
    nch = bpw // ch
    assert b % _NW == 0 and bpw % ch == 0
    mesh = plsc.VectorSubcoreMesh(core_axis_name="c", subcore_axis_name="s")

    @functools.partial(
        pl.kernel, mesh=mesh,
        compiler_params=pltpu.CompilerParams(use_tc_tiling_on_sc=tc_tiling),
        out_type=jax.ShapeDtypeStruct((b, d), jnp.float32),
        scratch_types=[
            pltpu.VMEM((ch,), jnp.int32),
            pltpu.VMEM((ch, d), jnp.float32),
            pltpu.SemaphoreType.DMA,
        ],
    )
    def k(table_hbm, idx_hbm, out_hbm, idx_v, rows_v, sem):
        wid = lax.axis_index("s") * _NC + lax.axis_index("c")
        base = wid * bpw

        def body(i, carry):
            off = base + i * ch
            pltpu.sync_copy(idx_hbm.at[pl.ds(off, ch)], idx_v)
            pltpu.async_copy(table_hbm.at[idx_v], rows_v, sem).wait()
            pltpu.sync_copy(rows_v, out_hbm.at[pl.ds(off, ch)])
            return carry

        lax.fori_loop(0, nch, body, 0)

    return k(table, idx)


def _sc_scatter(m, dst, zeros, d):
    e = m.shape[0]
    epw = e // _NW
    ch = 200
    nch = epw // ch
    rps = NPAD // _NS
    assert e % _NW == 0 and epw % ch == 0
    mesh = plsc.VectorSubcoreMesh(core_axis_name="c", subcore_axis_name="s")

    @functools.partial(
        pl.kernel, mesh=mesh,
        out_type=jax.ShapeDtypeStruct((_NC, NPAD, d), jnp.float32),
        scratch_types=[
            pltpu.VMEM((ch,), jnp.int32),
            pltpu.VMEM((ch, d), jnp.float32),
            pltpu.VMEM_SHARED((NPAD, d), jnp.float32),
            pltpu.SemaphoreType.DMA,
        ],
    )
    def k(m_hbm, dst_hbm, zero_hbm, out_hbm, idx_v, m_v, accum, sem):
        cid = lax.axis_index("c")
        sid = lax.axis_index("s")
        wid = sid * _NC + cid
        pltpu.sync_copy(zero_hbm.at[pl.ds(sid * rps, rps)],
                        accum.at[pl.ds(sid * rps, rps)])
        plsc.subcore_barrier()
        base = wid * epw

        def body(i, carry):
            off = base + i * ch
            pltpu.sync_copy(dst_hbm.at[pl.ds(off, ch)], idx_v)
            pltpu.sync_copy(m_hbm.at[pl.ds(off, ch)], m_v)
            pltpu.sync_copy(m_v, accum.at[idx_v], add=True)
            return carry

        lax.fori_loop(0, nch, body, 0)
        plsc.subcore_barrier()
        pltpu.sync_copy(accum.at[pl.ds(sid * rps, rps)],
                        out_hbm.at[cid, pl.ds(sid * rps, rps)])

    return k(m, dst, zeros)


def _premlp_body(x_ref, w0, b0, w1, b1, w2, b2, w3, b3, o_ref):
    h = x_ref[...]
    for w, b in ((w0, b0), (w1, b1), (w2, b2), (w3, b3)):
        h = jnp.maximum(jnp.dot(h, w[...],
                                preferred_element_type=jnp.float32) + b[...], 0.0)
    o_ref[...] = h


def _premlp(x, ws, bs):
    args = []
    for w, b in zip(ws, bs):
        args += [w, b]
    inspecs = [pl.BlockSpec((N, 128), lambda i: (0, 0))]
    for w, b in zip(ws, bs):
        inspecs.append(pl.BlockSpec(w.shape, lambda i: (0, 0)))
        inspecs.append(pl.BlockSpec(b.shape, lambda i: (0, 0)))
    return pl.pallas_call(
        _premlp_body,
        grid=(1,),
        in_specs=inspecs,
        out_specs=pl.BlockSpec((N, DT), lambda i: (0, 0)),
        out_shape=jax.ShapeDtypeStruct((N, DT), jnp.float32),
    )(x, *args)


def _proj_body(x_ref, wf_ref, bf_ref, ws_ref, bs_ref, tf_ref, ts_ref):
    x = x_ref[...]
    wf = wf_ref[...].reshape(DT, DT)
    ws = ws_ref[...].reshape(DT, DT)
    tf_ref[...] = (jnp.dot(x, wf, preferred_element_type=jnp.float32)
                   + bf_ref[0])[None]
    ts_ref[...] = (jnp.dot(x, ws, preferred_element_type=jnp.float32)
                   + bs_ref[0])[None]


def _proj(xc, wf_stack, bf_stack, ws_stack, bs_stack):
    blk = 2000
    grid = (2, N // blk)
    out_sh = jax.ShapeDtypeStruct((2, N, DT), jnp.float32)
    return pl.pallas_call(
        _proj_body,
        grid=grid,
        in_specs=[
            pl.BlockSpec((blk, DT), lambda h, i: (i, 0)),
            pl.BlockSpec((1, DT, DT), lambda h, i: (h, 0, 0)),
            pl.BlockSpec((1, 1, DT), lambda h, i: (h, 0, 0)),
            pl.BlockSpec((1, DT, DT), lambda h, i: (h, 0, 0)),
            pl.BlockSpec((1, 1, DT), lambda h, i: (h, 0, 0)),
        ],
        out_specs=[
            pl.BlockSpec((1, blk, DT), lambda h, i: (h, i, 0)),
            pl.BlockSpec((1, blk, DT), lambda h, i: (h, i, 0)),
        ],
        out_shape=[out_sh, out_sh],
    )(xc, wf_stack, bf_stack, ws_stack, bs_stack)


def _edge_body(gfd, gfs, gsd, gss, ea_ref, wfe_ref, wse_ref, o_ref):
    ea = ea_ref[...]
    f = gfd[...] + gfs[...] + jnp.dot(ea, wfe_ref[...],
                                      preferred_element_type=jnp.float32)
    s = gsd[...] + gss[...] + jnp.dot(ea, wse_ref[...],
                                      preferred_element_type=jnp.float32)
    sig = 1.0 / (1.0 + jnp.exp(-f))
    sp = jnp.maximum(s, 0.0) + jnp.log1p(jnp.exp(-jnp.abs(s)))
    m = sig * sp
    li = lax.broadcasted_iota(jnp.int32, m.shape, 1)
    o_ref[...] = jnp.where(li < D1, m, jnp.where(li == D1, 1.0, 0.0))


def _edge(gf, gs, ea, wfe, wse):
    e = ea.shape[0]
    a = ea.shape[1]
    blk = 4000
    nblk = e // blk
    assert e % blk == 0
    return pl.pallas_call(
        _edge_body,
        grid=(nblk,),
        in_specs=[
            pl.BlockSpec((blk, DT), lambda i: (i, 0)),
            pl.BlockSpec((blk, DT), lambda i: (i + nblk, 0)),
            pl.BlockSpec((blk, DT), lambda i: (i, 0)),
            pl.BlockSpec((blk, DT), lambda i: (i + nblk, 0)),
            pl.BlockSpec((blk, a), lambda i: (i, 0)),
            pl.BlockSpec((a, DT), lambda i: (0, 0)),
            pl.BlockSpec((a, DT), lambda i: (0, 0)),
        ],
        out_specs=pl.BlockSpec((blk, DT), lambda i: (i, 0)),
        out_shape=jax.ShapeDtypeStruct((e, DT), jnp.float32),
    )(gf, gf, gs, gs, ea, wfe, wse)


def _post_body(parts_ref, xc_ref, g_ref, b_ref, o_ref):
    s2 = parts_ref[0] + parts_ref[1]
    li = lax.broadcasted_iota(jnp.int32, s2.shape, 1)
    cnt = jnp.sum(jnp.where(li == D1, s2, 0.0), axis=1, keepdims=True)
    upd = xc_ref[...] + jnp.where(li < D1, s2, 0.0) / jnp.maximum(cnt, 1.0)
    mu = jnp.mean(upd, axis=0, keepdims=True)
    v = jnp.mean((upd - mu) ** 2, axis=0, keepdims=True)
    hn = (upd - mu) / jnp.sqrt(v + 1e-5) * g_ref[...] + b_ref[...]
    o_ref[...] = jnp.where(li < D1, hn, 0.0)


def _post(parts, xc, g, b):
    return pl.pallas_call(
        _post_body,
        grid=(1,),
        in_specs=[
            pl.BlockSpec((2, N, DT), lambda i: (0, 0, 0)),
            pl.BlockSpec((N, DT), lambda i: (0, 0)),
            pl.BlockSpec((1, DT), lambda i: (0, 0)),
            pl.BlockSpec((1, DT), lambda i: (0, 0)),
        ],
        out_specs=pl.BlockSpec((N, DT), lambda i: (0, 0)),
        out_shape=jax.ShapeDtypeStruct((N, DT), jnp.float32),
    )(parts, xc, g, b)


def _post_score_body(parts_ref, xc_ref, g_ref, b_ref, pwa_ref, pwb_ref, o_ref):
    s2 = parts_ref[0] + parts_ref[1]
    li = lax.broadcasted_iota(jnp.int32, s2.shape, 1)
    cnt = jnp.sum(jnp.where(li == D1, s2, 0.0), axis=1, keepdims=True)
    upd = xc_ref[...] + jnp.where(li < D1, s2, 0.0) / jnp.maximum(cnt, 1.0)
    mu = jnp.mean(upd, axis=0, keepdims=True)
    v = jnp.mean((upd - mu) ** 2, axis=0, keepdims=True)
    xi = (upd - mu) / jnp.sqrt(v + 1e-5) * g_ref[...] + b_ref[...]
    xi = jnp.where(li < D1, xi, 0.0)
    o_ref[0] = jnp.dot(xi, pwb_ref[...], preferred_element_type=jnp.float32)
    o_ref[1] = jnp.dot(xi, pwa_ref[...], preferred_element_type=jnp.float32)


def _post_score(parts, xc, g, b, pwa, pwb):
    return pl.pallas_call(
        _post_score_body,
        grid=(1,),
        in_specs=[
            pl.BlockSpec((2, N, DT), lambda i: (0, 0, 0)),
            pl.BlockSpec((N, DT), lambda i: (0, 0)),
            pl.BlockSpec((1, DT), lambda i: (0, 0)),
            pl.BlockSpec((1, DT), lambda i: (0, 0)),
            pl.BlockSpec((DT, 16), lambda i: (0, 0)),
            pl.BlockSpec((DT, 16), lambda i: (0, 0)),
        ],
        out_specs=pl.BlockSpec((2, N, 16), lambda i: (0, 0, 0)),
        out_shape=jax.ShapeDtypeStruct((2, N, 16), jnp.float32),
    )(parts, xc, g, b, pwa, pwb)


def _ea4_body(sgd, sgs, ea_ref, pwc_ref, pb_ref, o_ref):
    ea = ea_ref[...]
    c = jnp.dot(ea, pwc_ref[...], preferred_element_type=jnp.float32)
    tot = sgd[...] + sgs[...] + c + pb_ref[...]
    sig = 1.0 / (1.0 + jnp.exp(-tot))
    li = lax.broadcasted_iota(jnp.int32, sig.shape, 1)
    sc = jnp.sum(jnp.where(li == 0, sig, 0.0), axis=1, keepdims=True)
    o_ref[...] = jnp.concatenate([ea, sc], axis=1)


def _ea4(sg, ea, pwc, pb16):
    e = ea.shape[0]
    blk = 4000
    nblk = e // blk
    return pl.pallas_call(
        _ea4_body,
        grid=(nblk,),
        in_specs=[
            pl.BlockSpec((blk, 16), lambda i: (i, 0)),
            pl.BlockSpec((blk, 16), lambda i: (i + nblk, 0)),
            pl.BlockSpec((blk, 3), lambda i: (i, 0)),
            pl.BlockSpec((3, 16), lambda i: (0, 0)),
            pl.BlockSpec((1, 16), lambda i: (0, 0)),
        ],
        out_specs=pl.BlockSpec((blk, 4), lambda i: (i, 0)),
        out_shape=jax.ShapeDtypeStruct((e, 4), jnp.float32),
    )(sg, sg, ea, pwc, pb16)


def _readout_body(out_ref, oh_ref, pars_ref, g1w_ref, g1wp_ref, g1b_ref,
                  g2w_ref, g2b_ref, o_ref):
    oh = oh_ref[...]
    ssum = lax.dot_general(oh, out_ref[...], (((0,), (0,)), ((), ())),
                           preferred_element_type=jnp.float32)
    cnt = jnp.sum(oh, axis=0)[:, None]
    pooled = ssum / jnp.maximum(cnt, 1.0)
    o1 = (jnp.dot(pooled, g1w_ref[...], preferred_element_type=jnp.float32)
          + jnp.dot(pars_ref[...], g1wp_ref[...],
                    preferred_element_type=jnp.float32) + g1b_ref[...])
    o_ref[...] = jnp.dot(o1, g2w_ref[...],
                         preferred_element_type=jnp.float32) + g2b_ref[...]


def _readout(out6, oh, pars, g1w, g1wp, g1b, g2w, g2b):
    g = oh.shape[1]
    return pl.pallas_call(
        _readout_body,
        grid=(1,),
        in_specs=[
            pl.BlockSpec((N, DT), lambda i: (0, 0)),
            pl.BlockSpec((N, g), lambda i: (0, 0)),
            pl.BlockSpec((g, 8), lambda i: (0, 0)),
            pl.BlockSpec((DT, 128), lambda i: (0, 0)),
            pl.BlockSpec((8, 128), lambda i: (0, 0)),
            pl.BlockSpec((1, 128), lambda i: (0, 0)),
            pl.BlockSpec((128, 8), lambda i: (0, 0)),
            pl.BlockSpec((1, 8), lambda i: (0, 0)),
        ],
        out_specs=pl.BlockSpec((g, 8), lambda i: (0, 0)),
        out_shape=jax.ShapeDtypeStruct((g, 8), jnp.float32),
    )(out6, oh, pars, g1w, g1wp, g1b, g2w, g2b)


def _padw(w, rows, cols):
    return jnp.pad(w, ((0, rows - w.shape[0]), (0, cols - w.shape[1])))


def _stackw(wf, bf):
    wd = _padw(wf[:D1], DT, DT)
    ws = _padw(wf[D1:2 * D1], DT, DT)
    w_stack = jnp.stack([wd, ws])
    b_stack = jnp.stack([jnp.pad(bf, (0, DT - D1)),
                         jnp.zeros((DT,), jnp.float32)]).reshape(2, 1, DT)
    return w_stack, b_stack


def _conv(xc, idx2e, dst, ea, wfe, wse, wf_stack, bf_stack, ws_stack, bs_stack,
          zeros):
    tf, ts = _proj(xc, wf_stack, bf_stack, ws_stack, bs_stack)
    tf = tf.reshape(2 * N, DT)
    ts = ts.reshape(2 * N, DT)
    gf = _sc_gather(tf, idx2e, DT)
    gs = _sc_gather(ts, idx2e, DT)
    m = _edge(gf, gs, ea, wfe, wse)
    return _sc_scatter(m, dst, zeros, DT)


def kernel(x, edge_index_1, edge_index_2, edge_index_3, edge_attr_1, edge_attr_2, edge_attr_3, batch, parameters, pre_W0, pre_b0, pre_W1, pre_b1, pre_W2, pre_b2, pre_W3, pre_b3, c1_Wf, c1_bf, c1_Ws, c1_bs, c12_Wf, c12_bf, c12_Ws, c12_bs, c13_Wf, c13_bf, c13_Ws, c13_bs, c14_Wf, c14_bf, c14_Ws, c14_bs, c24_Wf, c24_bf, c24_Ws, c24_bs, c34_Wf, c34_bf, c34_Ws, c34_bs, pool_W, pool_b, bn1_g, bn1_b, bn12_g, bn12_b, bn13_g, bn13_b, bn14_g, bn14_b, bn24_g, bn24_b, bn34_g, bn34_b, g1_W, g1_b, g2_W, g2_b):
    f32 = jnp.float32
    zeros = jnp.zeros((NPAD, DT), f32)

    ws = [_padw(pre_W0, 128, DT)] + [_padw(w, DT, DT)
                                     for w in (pre_W1, pre_W2, pre_W3)]
    bs = [jnp.pad(b, (0, DT - D1)).reshape(1, DT)
          for b in (pre_b0, pre_b1, pre_b2, pre_b3)]
    h = _premlp(x, ws, bs)

    pwa = _padw(pool_W[:D1], DT, 16)
    pwb = _padw(pool_W[D1:2 * D1], DT, 16)
    pwc = _padw(pool_W[2 * D1:2 * D1 + 3], 3, 16)
    pb16 = jnp.pad(pool_b, (0, 15)).reshape(1, 16)

    small = [
        (edge_index_1, edge_attr_1, c1_Wf, c1_bf, c1_Ws, c1_bs, bn1_g, bn1_b),
        (edge_index_2, edge_attr_2, c12_Wf, c12_bf, c12_Ws, c12_bs, bn12_g, bn12_b),
        (edge_index_3, edge_attr_3, c13_Wf, c13_bf, c13_Ws, c13_bs, bn13_g, bn13_b),
    ]
    ea4s = []
    for ei, ea, wf, bf, wsm, bsm, g, b in small:
        src, dst = ei[0], ei[1]
        idx2e = jnp.concatenate([dst, src + N])
        wf_stack, bf_stack = _stackw(wf, bf)
        ws_stack, bs_stack = _stackw(wsm, bsm)
        wfe = _padw(wf[2 * D1:], 3, DT)
        wse = _padw(wsm[2 * D1:], 3, DT)
        parts = _conv(h, idx2e, dst, ea, wfe, wse,
                      wf_stack, bf_stack, ws_stack, bs_stack, zeros)
        gp = jnp.pad(g, (0, DT - D1)).reshape(1, DT)
        bp = jnp.pad(b, (0, DT - D1)).reshape(1, DT)
        stab = _post_score(parts, h, gp, bp, pwa, pwb)
        sg = _sc_gather(stab.reshape(2 * N, 16), idx2e, 16, tc_tiling=False)
        ea4s.append(_ea4(sg, ea, pwc, pb16))

    ei_big = jnp.concatenate([edge_index_1, edge_index_2, edge_index_3], axis=1)
    ea4 = jnp.concatenate(ea4s, axis=0)
    src_b, dst_b = ei_big[0], ei_big[1]
    idx2e_b = jnp.concatenate([dst_b, src_b + N])

    out = h
    for wf, bf, wsm, bsm, g, b in (
        (c14_Wf, c14_bf, c14_Ws, c14_bs, bn14_g, bn14_b),
        (c24_Wf, c24_bf, c24_Ws, c24_bs, bn24_g, bn24_b),
        (c34_Wf, c34_bf, c34_Ws, c34_bs, bn34_g, bn34_b),
    ):
        wf_stack, bf_stack = _stackw(wf, bf)
        ws_stack, bs_stack = _stackw(wsm, bsm)
        wfe = _padw(wf[2 * D1:], 4, DT)
        wse = _padw(wsm[2 * D1:], 4, DT)
        parts = _conv(out, idx2e_b, dst_b, ea4, wfe, wse,
                      wf_stack, bf_stack, ws_stack, bs_stack, zeros)
        gp = jnp.pad(g, (0, DT - D1)).reshape(1, DT)
        bp = jnp.pad(b, (0, DT - D1)).reshape(1, DT)
        out = _post(parts, out, gp, bp)

    gcount = parameters.shape[0]
    oh = jax.nn.one_hot(batch, gcount, dtype=f32)
    pars = jnp.pad(parameters, ((0, 0), (0, 6)))
    g1w = _padw(g1_W[:D1], DT, 128)
    g1wp = _padw(g1_W[D1:D1 + 2], 8, 128)
    g1b = g1_b.reshape(1, 128)
    g2w = _padw(g2_W, 128, 8)
    g2b = jnp.pad(g2_b, (0, 7)).reshape(1, 8)
    o = _readout(out, oh, pars, g1w, g1wp, g1b, g2w, g2b)
    return o[:, 0]


# speedup vs baseline: 5.2492x; 1.1608x over previous
import functools

import jax
import jax.numpy as jnp
from jax import lax
from jax.experimental import pallas as pl
from jax.experimental.pallas import tpu as pltpu
from jax.experimental.pallas import tpu_sc as plsc

N = 10000
NPAD = 10240
D1 = 100
DT = 128
_NC, _NS = 2, 16
_NW = _NC * _NS


def _sc_gather(table, idx, d, tc_tiling=True):
    b = idx.shape[0]
    bpw = b // _NW
    ch = 400
    nch = bpw // ch
    assert b % _NW == 0 and bpw % ch == 0
    mesh = plsc.VectorSubcoreMesh(core_axis_name="c", subcore_axis_name="s")

    @functools.partial(
        pl.kernel, mesh=mesh,
        compiler_params=pltpu.CompilerParams(use_tc_tiling_on_sc=tc_tiling),
        out_type=jax.ShapeDtypeStruct((b, d), jnp.float32),
        scratch_types=[
            pltpu.VMEM((ch,), jnp.int32),
            pltpu.VMEM((ch, d), jnp.float32),
            pltpu.SemaphoreType.DMA,
        ],
    )
    def k(table_hbm, idx_hbm, out_hbm, idx_v, rows_v, sem):
        wid = lax.axis_index("s") * _NC + lax.axis_index("c")
        base = wid * bpw

        def body(i, carry):
            off = base + i * ch
            pltpu.sync_copy(idx_hbm.at[pl.ds(off, ch)], idx_v)
            pltpu.async_copy(table_hbm.at[idx_v], rows_v, sem).wait()
            pltpu.sync_copy(rows_v, out_hbm.at[pl.ds(off, ch)])
            return carry

        lax.fori_loop(0, nch, body, 0)

    return k(table, idx)


def _sc_gather_pair(tf, ts, idx2e, d):
    e = idx2e.shape[0] // 2
    epw = e // _NW
    ch = 200
    nch = epw // ch
    assert e % _NW == 0 and epw % ch == 0
    mesh = plsc.VectorSubcoreMesh(core_axis_name="c", subcore_axis_name="s")
    out_sh = jax.ShapeDtypeStruct((e, d), jnp.float32)

    @functools.partial(
        pl.kernel, mesh=mesh,
        compiler_params=pltpu.CompilerParams(use_tc_tiling_on_sc=True),
        out_type=(out_sh, out_sh),
        scratch_types=[
            pltpu.VMEM((ch,), jnp.int32),
            pltpu.VMEM((ch,), jnp.int32),
            pltpu.VMEM((ch, d), jnp.float32),
            pltpu.VMEM((ch, d), jnp.float32),
            pltpu.SemaphoreType.DMA,
            pltpu.SemaphoreType.DMA,
        ],
    )
    def k(tf_hbm, ts_hbm, idx_hbm, gf_hbm, gs_hbm,
          idxd_v, idxs_v, rf_v, rs_v, semf, sems):
        wid = lax.axis_index("s") * _NC + lax.axis_index("c")
        base = wid * epw

        def body(i, carry):
            off = base + i * ch
            pltpu.sync_copy(idx_hbm.at[pl.ds(off, ch)], idxd_v)
            pltpu.sync_copy(idx_hbm.at[pl.ds(e + off, ch)], idxs_v)
            cf = pltpu.async_copy(tf_hbm.at[idxd_v], rf_v, semf)
            cs = pltpu.async_copy(ts_hbm.at[idxd_v], rs_v, sems)
            cf.wait()
            pltpu.async_copy(tf_hbm.at[idxs_v], rf_v, semf, add=True).wait()
            cs.wait()
            pltpu.async_copy(ts_hbm.at[idxs_v], rs_v, sems, add=True).wait()
            pltpu.sync_copy(rf_v, gf_hbm.at[pl.ds(off, ch)])
            pltpu.sync_copy(rs_v, gs_hbm.at[pl.ds(off, ch)])
            return carry

        lax.fori_loop(0, nch, body, 0)

    return k(tf, ts, idx2e)


def _sc_scatter(m, dst, zeros, d):
    e = m.shape[0]
    epw = e // _NW
    ch = 200
    nch = epw // ch
    rps = NPAD // _NS
    assert e % _NW == 0 and epw % ch == 0
    mesh = plsc.VectorSubcoreMesh(core_axis_name="c", subcore_axis_name="s")

    @functools.partial(
        pl.kernel, mesh=mesh,
        out_type=jax.ShapeDtypeStruct((_NC, NPAD, d), jnp.float32),
        scratch_types=[
            pltpu.VMEM((ch,), jnp.int32),
            pltpu.VMEM((ch, d), jnp.float32),
            pltpu.VMEM_SHARED((NPAD, d), jnp.float32),
            pltpu.SemaphoreType.DMA,
        ],
    )
    def k(m_hbm, dst_hbm, zero_hbm, out_hbm, idx_v, m_v, accum, sem):
        cid = lax.axis_index("c")
        sid = lax.axis_index("s")
        wid = sid * _NC + cid
        pltpu.sync_copy(zero_hbm.at[pl.ds(sid * rps, rps)],
                        accum.at[pl.ds(sid * rps, rps)])
        plsc.subcore_barrier()
        base = wid * epw

        def body(i, carry):
            off = base + i * ch
            pltpu.sync_copy(dst_hbm.at[pl.ds(off, ch)], idx_v)
            pltpu.sync_copy(m_hbm.at[pl.ds(off, ch)], m_v)
            pltpu.sync_copy(m_v, accum.at[idx_v], add=True)
            return carry

        lax.fori_loop(0, nch, body, 0)
        plsc.subcore_barrier()
        pltpu.sync_copy(accum.at[pl.ds(sid * rps, rps)],
                        out_hbm.at[cid, pl.ds(sid * rps, rps)])

    return k(m, dst, zeros)


def _premlp_body(x_ref, w0, b0, w1, b1, w2, b2, w3, b3, o_ref):
    h = x_ref[...]
    for w, b in ((w0, b0), (w1, b1), (w2, b2), (w3, b3)):
        h = jnp.maximum(jnp.dot(h, w[...],
                                preferred_element_type=jnp.float32) + b[...], 0.0)
    o_ref[...] = h


def _premlp(x, ws, bs):
    args = []
    for w, b in zip(ws, bs):
        args += [w, b]
    inspecs = [pl.BlockSpec((N, 128), lambda i: (0, 0))]
    for w, b in zip(ws, bs):
        inspecs.append(pl.BlockSpec(w.shape, lambda i: (0, 0)))
        inspecs.append(pl.BlockSpec(b.shape, lambda i: (0, 0)))
    return pl.pallas_call(
        _premlp_body,
        grid=(1,),
        in_specs=inspecs,
        out_specs=pl.BlockSpec((N, DT), lambda i: (0, 0)),
        out_shape=jax.ShapeDtypeStruct((N, DT), jnp.float32),
    )(x, *args)


def _proj_body(x_ref, wf_ref, bf_ref, ws_ref, bs_ref, tf_ref, ts_ref):
    x = x_ref[...]
    wf = wf_ref[...].reshape(DT, DT)
    ws = ws_ref[...].reshape(DT, DT)
    tf_ref[...] = (jnp.dot(x, wf, preferred_element_type=jnp.float32)
                   + bf_ref[0])[None]
    ts_ref[...] = (jnp.dot(x, ws, preferred_element_type=jnp.float32)
                   + bs_ref[0])[None]


def _proj(xc, wf_stack, bf_stack, ws_stack, bs_stack):
    blk = 2000
    grid = (2, N // blk)
    out_sh = jax.ShapeDtypeStruct((2, N, DT), jnp.float32)
    return pl.pallas_call(
        _proj_body,
        grid=grid,
        in_specs=[
            pl.BlockSpec((blk, DT), lambda h, i: (i, 0)),
            pl.BlockSpec((1, DT, DT), lambda h, i: (h, 0, 0)),
            pl.BlockSpec((1, 1, DT), lambda h, i: (h, 0, 0)),
            pl.BlockSpec((1, DT, DT), lambda h, i: (h, 0, 0)),
            pl.BlockSpec((1, 1, DT), lambda h, i: (h, 0, 0)),
        ],
        out_specs=[
            pl.BlockSpec((1, blk, DT), lambda h, i: (h, i, 0)),
            pl.BlockSpec((1, blk, DT), lambda h, i: (h, i, 0)),
        ],
        out_shape=[out_sh, out_sh],
    )(xc, wf_stack, bf_stack, ws_stack, bs_stack)


def _edge_body(gf, gs, ea_ref, wfe_ref, wse_ref, o_ref):
    ea = ea_ref[...]
    f = gf[...] + jnp.dot(ea, wfe_ref[...],
                          preferred_element_type=jnp.float32)
    s = gs[...] + jnp.dot(ea, wse_ref[...],
                          preferred_element_type=jnp.float32)
    sig = 1.0 / (1.0 + jnp.exp(-f))
    sp = jnp.maximum(s, 0.0) + jnp.log1p(jnp.exp(-jnp.abs(s)))
    m = sig * sp
    li = lax.broadcasted_iota(jnp.int32, m.shape, 1)
    o_ref[...] = jnp.where(li < D1, m, jnp.where(li == D1, 1.0, 0.0))


def _edge(gf, gs, ea, wfe, wse):
    e = ea.shape[0]
    a = ea.shape[1]
    blk = 4000
    nblk = e // blk
    assert e % blk == 0
    return pl.pallas_call(
        _edge_body,
        grid=(nblk,),
        in_specs=[
            pl.BlockSpec((blk, DT), lambda i: (i, 0)),
            pl.BlockSpec((blk, DT), lambda i: (i, 0)),
            pl.BlockSpec((blk, a), lambda i: (i, 0)),
            pl.BlockSpec((a, DT), lambda i: (0, 0)),
            pl.BlockSpec((a, DT), lambda i: (0, 0)),
        ],
        out_specs=pl.BlockSpec((blk, DT), lambda i: (i, 0)),
        out_shape=jax.ShapeDtypeStruct((e, DT), jnp.float32),
    )(gf, gs, ea, wfe, wse)


def _post_body(parts_ref, xc_ref, g_ref, b_ref, o_ref):
    s2 = parts_ref[0] + parts_ref[1]
    li = lax.broadcasted_iota(jnp.int32, s2.shape, 1)
    cnt = jnp.sum(jnp.where(li == D1, s2, 0.0), axis=1, keepdims=True)
    upd = xc_ref[...] + jnp.where(li < D1, s2, 0.0) / jnp.maximum(cnt, 1.0)
    mu = jnp.mean(upd, axis=0, keepdims=True)
    v = jnp.mean((upd - mu) ** 2, axis=0, keepdims=True)
    hn = (upd - mu) / jnp.sqrt(v + 1e-5) * g_ref[...] + b_ref[...]
    o_ref[...] = jnp.where(li < D1, hn, 0.0)


def _post(parts, xc, g, b):
    return pl.pallas_call(
        _post_body,
        grid=(1,),
        in_specs=[
            pl.BlockSpec((2, N, DT), lambda i: (0, 0, 0)),
            pl.BlockSpec((N, DT), lambda i: (0, 0)),
            pl.BlockSpec((1, DT), lambda i: (0, 0)),
            pl.BlockSpec((1, DT), lambda i: (0, 0)),
        ],
        out_specs=pl.BlockSpec((N, DT), lambda i: (0, 0)),
        out_shape=jax.ShapeDtypeStruct((N, DT), jnp.float32),
    )(parts, xc, g, b)


def _post_score_body(parts_ref, xc_ref, g_ref, b_ref, pwa_ref, pwb_ref, o_ref):
    s2 = parts_ref[0] + parts_ref[1]
    li = lax.broadcasted_iota(jnp.int32, s2.shape, 1)
    cnt = jnp.sum(jnp.where(li == D1, s2, 0.0), axis=1, keepdims=True)
    upd = xc_ref[...] + jnp.where(li < D1, s2, 0.0) / jnp.maximum(cnt, 1.0)
    mu = jnp.mean(upd, axis=0, keepdims=True)
    v = jnp.mean((upd - mu) ** 2, axis=0, keepdims=True)
    xi = (upd - mu) / jnp.sqrt(v + 1e-5) * g_ref[...] + b_ref[...]
    xi = jnp.where(li < D1, xi, 0.0)
    o_ref[0] = jnp.dot(xi, pwb_ref[...], preferred_element_type=jnp.float32)
    o_ref[1] = jnp.dot(xi, pwa_ref[...], preferred_element_type=jnp.float32)


def _post_score(parts, xc, g, b, pwa, pwb):
    return pl.pallas_call(
        _post_score_body,
        grid=(1,),
        in_specs=[
            pl.BlockSpec((2, N, DT), lambda i: (0, 0, 0)),
            pl.BlockSpec((N, DT), lambda i: (0, 0)),
            pl.BlockSpec((1, DT), lambda i: (0, 0)),
            pl.BlockSpec((1, DT), lambda i: (0, 0)),
            pl.BlockSpec((DT, 16), lambda i: (0, 0)),
            pl.BlockSpec((DT, 16), lambda i: (0, 0)),
        ],
        out_specs=pl.BlockSpec((2, N, 16), lambda i: (0, 0, 0)),
        out_shape=jax.ShapeDtypeStruct((2, N, 16), jnp.float32),
    )(parts, xc, g, b, pwa, pwb)


def _ea4_body(sgd, sgs, ea_ref, pwc_ref, pb_ref, o_ref):
    ea = ea_ref[...]
    c = jnp.dot(ea, pwc_ref[...], preferred_element_type=jnp.float32)
    tot = sgd[...] + sgs[...] + c + pb_ref[...]
    sig = 1.0 / (1.0 + jnp.exp(-tot))
    li = lax.broadcasted_iota(jnp.int32, sig.shape, 1)
    sc = jnp.sum(jnp.where(li == 0, sig, 0.0), axis=1, keepdims=True)
    o_ref[...] = jnp.concatenate([ea, sc], axis=1)


def _ea4(sg, ea, pwc, pb16):
    e = ea.shape[0]
    blk = 4000
    nblk = e // blk
    return pl.pallas_call(
        _ea4_body,
        grid=(nblk,),
        in_specs=[
            pl.BlockSpec((blk, 16), lambda i: (i, 0)),
            pl.BlockSpec((blk, 16), lambda i: (i + nblk, 0)),
            pl.BlockSpec((blk, 3), lambda i: (i, 0)),
            pl.BlockSpec((3, 16), lambda i: (0, 0)),
            pl.BlockSpec((1, 16), lambda i: (0, 0)),
        ],
        out_specs=pl.BlockSpec((blk, 4), lambda i: (i, 0)),
        out_shape=jax.ShapeDtypeStruct((e, 4), jnp.float32),
    )(sg, sg, ea, pwc, pb16)


def _readout_body(out_ref, oh_ref, pars_ref, g1w_ref, g1wp_ref, g1b_ref,
                  g2w_ref, g2b_ref, o_ref):
    oh = oh_ref[...]
    ssum = lax.dot_general(oh, out_ref[...], (((0,), (0,)), ((), ())),
                           preferred_element_type=jnp.float32)
    cnt = jnp.sum(oh, axis=0)[:, None]
    pooled = ssum / jnp.maximum(cnt, 1.0)
    o1 = (jnp.dot(pooled, g1w_ref[...], preferred_element_type=jnp.float32)
          + jnp.dot(pars_ref[...], g1wp_ref[...],
                    preferred_element_type=jnp.float32) + g1b_ref[...])
    o_ref[...] = jnp.dot(o1, g2w_ref[...],
                         preferred_element_type=jnp.float32) + g2b_ref[...]


def _readout(out6, oh, pars, g1w, g1wp, g1b, g2w, g2b):
    g = oh.shape[1]
    return pl.pallas_call(
        _readout_body,
        grid=(1,),
        in_specs=[
            pl.BlockSpec((N, DT), lambda i: (0, 0)),
            pl.BlockSpec((N, g), lambda i: (0, 0)),
            pl.BlockSpec((g, 8), lambda i: (0, 0)),
            pl.BlockSpec((DT, 128), lambda i: (0, 0)),
            pl.BlockSpec((8, 128), lambda i: (0, 0)),
            pl.BlockSpec((1, 128), lambda i: (0, 0)),
            pl.BlockSpec((128, 8), lambda i: (0, 0)),
            pl.BlockSpec((1, 8), lambda i: (0, 0)),
        ],
        out_specs=pl.BlockSpec((g, 8), lambda i: (0, 0)),
        out_shape=jax.ShapeDtypeStruct((g, 8), jnp.float32),
    )(out6, oh, pars, g1w, g1wp, g1b, g2w, g2b)


def _padw(w, rows, cols):
    return jnp.pad(w, ((0, rows - w.shape[0]), (0, cols - w.shape[1])))


def _stackw(wf, bf):
    wd = _padw(wf[:D1], DT, DT)
    ws = _padw(wf[D1:2 * D1], DT, DT)
    w_stack = jnp.stack([wd, ws])
    b_stack = jnp.stack([jnp.pad(bf, (0, DT - D1)),
                         jnp.zeros((DT,), jnp.float32)]).reshape(2, 1, DT)
    return w_stack, b_stack


def _conv(xc, idx2e, dst, ea, wfe, wse, wf_stack, bf_stack, ws_stack, bs_stack,
          zeros):
    tf, ts = _proj(xc, wf_stack, bf_stack, ws_stack, bs_stack)
    tf = tf.reshape(2 * N, DT)
    ts = ts.reshape(2 * N, DT)
    gf, gs = _sc_gather_pair(tf, ts, idx2e, DT)
    m = _edge(gf, gs, ea, wfe, wse)
    return _sc_scatter(m, dst, zeros, DT)


def kernel(x, edge_index_1, edge_index_2, edge_index_3, edge_attr_1, edge_attr_2, edge_attr_3, batch, parameters, pre_W0, pre_b0, pre_W1, pre_b1, pre_W2, pre_b2, pre_W3, pre_b3, c1_Wf, c1_bf, c1_Ws, c1_bs, c12_Wf, c12_bf, c12_Ws, c12_bs, c13_Wf, c13_bf, c13_Ws, c13_bs, c14_Wf, c14_bf, c14_Ws, c14_bs, c24_Wf, c24_bf, c24_Ws, c24_bs, c34_Wf, c34_bf, c34_Ws, c34_bs, pool_W, pool_b, bn1_g, bn1_b, bn12_g, bn12_b, bn13_g, bn13_b, bn14_g, bn14_b, bn24_g, bn24_b, bn34_g, bn34_b, g1_W, g1_b, g2_W, g2_b):
    f32 = jnp.float32
    zeros = jnp.zeros((NPAD, DT), f32)

    ws = [_padw(pre_W0, 128, DT)] + [_padw(w, DT, DT)
                                     for w in (pre_W1, pre_W2, pre_W3)]
    bs = [jnp.pad(b, (0, DT - D1)).reshape(1, DT)
          for b in (pre_b0, pre_b1, pre_b2, pre_b3)]
    h = _premlp(x, ws, bs)

    pwa = _padw(pool_W[:D1], DT, 16)
    pwb = _padw(pool_W[D1:2 * D1], DT, 16)
    pwc = _padw(pool_W[2 * D1:2 * D1 + 3], 3, 16)
    pb16 = jnp.pad(pool_b, (0, 15)).reshape(1, 16)

    small = [
        (edge_index_1, edge_attr_1, c1_Wf, c1_bf, c1_Ws, c1_bs, bn1_g, bn1_b),
        (edge_index_2, edge_attr_2, c12_Wf, c12_bf, c12_Ws, c12_bs, bn12_g, bn12_b),
        (edge_index_3, edge_attr_3, c13_Wf, c13_bf, c13_Ws, c13_bs, bn13_g, bn13_b),
    ]
    ea4s = []
    for ei, ea, wf, bf, wsm, bsm, g, b in small:
        src, dst = ei[0], ei[1]
        idx2e = jnp.concatenate([dst, src + N])
        wf_stack, bf_stack = _stackw(wf, bf)
        ws_stack, bs_stack = _stackw(wsm, bsm)
        wfe = _padw(wf[2 * D1:], 3, DT)
        wse = _padw(wsm[2 * D1:], 3, DT)
        parts = _conv(h, idx2e, dst, ea, wfe, wse,
                      wf_stack, bf_stack, ws_stack, bs_stack, zeros)
        gp = jnp.pad(g, (0, DT - D1)).reshape(1, DT)
        bp = jnp.pad(b, (0, DT - D1)).reshape(1, DT)
        stab = _post_score(parts, h, gp, bp, pwa, pwb)
        sg = _sc_gather(stab.reshape(2 * N, 16), idx2e, 16, tc_tiling=False)
        ea4s.append(_ea4(sg, ea, pwc, pb16))

    ei_big = jnp.concatenate([edge_index_1, edge_index_2, edge_index_3], axis=1)
    ea4 = jnp.concatenate(ea4s, axis=0)
    src_b, dst_b = ei_big[0], ei_big[1]
    idx2e_b = jnp.concatenate([dst_b, src_b + N])

    out = h
    for wf, bf, wsm, bsm, g, b in (
        (c14_Wf, c14_bf, c14_Ws, c14_bs, bn14_g, bn14_b),
        (c24_Wf, c24_bf, c24_Ws, c24_bs, bn24_g, bn24_b),
        (c34_Wf, c34_bf, c34_Ws, c34_bs, bn34_g, bn34_b),
    ):
        wf_stack, bf_stack = _stackw(wf, bf)
        ws_stack, bs_stack = _stackw(wsm, bsm)
        wfe = _padw(wf[2 * D1:], 4, DT)
        wse = _padw(wsm[2 * D1:], 4, DT)
        parts = _conv(out, idx2e_b, dst_b, ea4, wfe, wse,
                      wf_stack, bf_stack, ws_stack, bs_stack, zeros)
        gp = jnp.pad(g, (0, DT - D1)).reshape(1, DT)
        bp = jnp.pad(b, (0, DT - D1)).reshape(1, DT)
        out = _post(parts, out, gp, bp)

    gcount = parameters.shape[0]
    oh = jax.nn.one_hot(batch, gcount, dtype=f32)
    pars = jnp.pad(parameters, ((0, 0), (0, 6)))
    g1w = _padw(g1_W[:D1], DT, 128)
    g1wp = _padw(g1_W[D1:D1 + 2], 8, 128)
    g1b = g1_b.reshape(1, 128)
    g2w = _padw(g2_W, 128, 8)
    g2b = jnp.pad(g2_b, (0, 7)).reshape(1, 8)
    o = _readout(out, oh, pars, g1w, g1wp, g1b, g2w, g2b)
    return o[:, 0]
